# Initial kernel scaffold; baseline (speedup 1.0000x reference)
import jax
import jax.numpy as jnp
from jax.experimental import pallas as pl


def kernel(input):
    raise NotImplementedError("write your pallas kernel here")


# speedup vs baseline: 1.4799x; 1.4799x over previous
import functools

import jax
import jax.numpy as jnp
from jax import lax
from jax.experimental import pallas as pl
from jax.experimental.pallas import tpu as pltpu
from jax.experimental.pallas import tpu_sc as plsc

WINDOW = 64
STRIDE = 32


def _sliding_window_sc(inp_hbm, out_hbm, buf):
    nc = 2
    wid = lax.axis_index("s") * nc + lax.axis_index("c")
    osz = out_hbm.shape[0]
    per = (osz + 31) // 32
    lo = wid * per

    for k in range(per):
        i = jnp.minimum(lo + k, osz - 1)
        pltpu.sync_copy(inp_hbm.at[pl.ds(i * STRIDE, WINDOW)], buf)
        pltpu.sync_copy(buf, out_hbm.at[i])


def kernel(input):
    T = input.shape[0]
    osz = (T - WINDOW) // STRIDE + 1
    D = input.shape[1]
    run = functools.partial(
        pl.kernel,
        mesh=plsc.VectorSubcoreMesh(core_axis_name="c", subcore_axis_name="s"),
        out_type=jax.ShapeDtypeStruct((osz, WINDOW, D), jnp.float32),
        scratch_types=[pltpu.VMEM((WINDOW, D), jnp.float32)],
    )(_sliding_window_sc)
    return run(input)


# speedup vs baseline: 1.7104x; 1.1557x over previous
import functools

import jax
import jax.numpy as jnp
from jax import lax
from jax.experimental import pallas as pl
from jax.experimental.pallas import tpu as pltpu
from jax.experimental.pallas import tpu_sc as plsc

WINDOW = 64
STRIDE = 32


def _sliding_window_sc(inp_hbm, out_hbm, buf):
    nc = 2
    wid = lax.axis_index("s") * nc + lax.axis_index("c")
    osz = out_hbm.shape[0]
    nblocks = inp_hbm.shape[0] // STRIDE
    per = nblocks // 32

    lo = wid * per
    for k in range(per):
        b = lo + k
        pltpu.sync_copy(inp_hbm.at[pl.ds(b * STRIDE, STRIDE)], buf)

        @pl.when(b < osz)
        def _():
            pltpu.sync_copy(buf, out_hbm.at[b, pl.ds(0, STRIDE)])

        @pl.when(b > 0)
        def _():
            pltpu.sync_copy(buf, out_hbm.at[b - 1, pl.ds(STRIDE, STRIDE)])


def kernel(input):
    T = input.shape[0]
    osz = (T - WINDOW) // STRIDE + 1
    D = input.shape[1]
    run = functools.partial(
        pl.kernel,
        mesh=plsc.VectorSubcoreMesh(core_axis_name="c", subcore_axis_name="s"),
        out_type=jax.ShapeDtypeStruct((osz, WINDOW, D), jnp.float32),
        scratch_types=[pltpu.VMEM((STRIDE, D), jnp.float32)],
    )(_sliding_window_sc)
    return run(input)


# speedup vs baseline: 2.0490x; 1.1980x over previous
import functools

import jax
import jax.numpy as jnp
from jax import lax
from jax.experimental import pallas as pl
from jax.experimental.pallas import tpu as pltpu
from jax.experimental.pallas import tpu_sc as plsc

WINDOW = 64
STRIDE = 32
NB = 4
AHEAD = 2


def _sliding_window_sc(inp_hbm, out_hbm, buf, rsems, wsems):
    nc = 2
    wid = lax.axis_index("s") * nc + lax.axis_index("c")
    osz = out_hbm.shape[0]
    nblocks = inp_hbm.shape[0] // STRIDE
    per = nblocks // 32

    lo = wid * per

    def read(k):
        slot = k % NB
        return pltpu.async_copy(
            inp_hbm.at[pl.ds((lo + k) * STRIDE, STRIDE)],
            buf.at[pl.ds(slot * STRIDE, STRIDE)],
            rsems[slot],
        )

    def write_descs(k):
        b = lo + k
        slot = k % NB
        src = buf.at[pl.ds(slot * STRIDE, STRIDE)]
        w1 = pltpu.make_async_copy(src, out_hbm.at[b, pl.ds(0, STRIDE)], wsems[slot])
        w2 = pltpu.make_async_copy(
            src, out_hbm.at[b - 1, pl.ds(STRIDE, STRIDE)], wsems[slot]
        )
        return b, w1, w2

    def fire_writes(k):
        b, w1, w2 = write_descs(k)

        @pl.when(b < osz)
        def _():
            w1.start()

        @pl.when(b > 0)
        def _():
            w2.start()

    def drain_writes(k):
        b, w1, w2 = write_descs(k)

        @pl.when(b < osz)
        def _():
            w1.wait()

        @pl.when(b > 0)
        def _():
            w2.wait()

    reads = {}
    for k in range(min(AHEAD, per)):
        reads[k] = read(k)
    for k in range(per):
        nk = k + AHEAD
        if nk < per:
            if nk - NB >= 0:
                drain_writes(nk - NB)
            reads[nk] = read(nk)
        reads[k].wait()
        fire_writes(k)
    for k in range(max(0, per - NB), per):
        drain_writes(k)


def kernel(input):
    T = input.shape[0]
    osz = (T - WINDOW) // STRIDE + 1
    D = input.shape[1]
    run = functools.partial(
        pl.kernel,
        mesh=plsc.VectorSubcoreMesh(core_axis_name="c", subcore_axis_name="s"),
        out_type=jax.ShapeDtypeStruct((osz, WINDOW, D), jnp.float32),
        scratch_types=[
            pltpu.VMEM((NB * STRIDE, D), jnp.float32),
            [pltpu.SemaphoreType.DMA] * NB,
            [pltpu.SemaphoreType.DMA] * NB,
        ],
    )(_sliding_window_sc)
    return run(input)


# speedup vs baseline: 2.0659x; 1.0083x over previous
import functools

import jax
import jax.numpy as jnp
from jax import lax
from jax.experimental import pallas as pl
from jax.experimental.pallas import tpu as pltpu
from jax.experimental.pallas import tpu_sc as plsc

WINDOW = 64
STRIDE = 32
NB = 6
AHEAD = 3


def _sliding_window_sc(inp_hbm, out_hbm, buf, rsems, wsems):
    nc = 2
    wid = lax.axis_index("s") * nc + lax.axis_index("c")
    osz = out_hbm.shape[0]
    nblocks = inp_hbm.shape[0] // STRIDE
    per = nblocks // 32

    lo = wid * per

    def read(k):
        slot = k % NB
        return pltpu.async_copy(
            inp_hbm.at[pl.ds((lo + k) * STRIDE, STRIDE)],
            buf.at[pl.ds(slot * STRIDE, STRIDE)],
            rsems[slot],
        )

    def write_descs(k):
        b = lo + k
        slot = k % NB
        src = buf.at[pl.ds(slot * STRIDE, STRIDE)]
        w1 = pltpu.make_async_copy(src, out_hbm.at[b, pl.ds(0, STRIDE)], wsems[slot])
        w2 = pltpu.make_async_copy(
            src, out_hbm.at[b - 1, pl.ds(STRIDE, STRIDE)], wsems[slot]
        )
        return b, w1, w2

    def fire_writes(k):
        b, w1, w2 = write_descs(k)

        @pl.when(b < osz)
        def _():
            w1.start()

        @pl.when(b > 0)
        def _():
            w2.start()

    def drain_writes(k):
        b, w1, w2 = write_descs(k)

        @pl.when(b < osz)
        def _():
            w1.wait()

        @pl.when(b > 0)
        def _():
            w2.wait()

    reads = {}
    for k in range(min(AHEAD, per)):
        reads[k] = read(k)
    for k in range(per):
        nk = k + AHEAD
        if nk < per:
            if nk - NB >= 0:
                drain_writes(nk - NB)
            reads[nk] = read(nk)
        reads[k].wait()
        fire_writes(k)
    for k in range(max(0, per - NB), per):
        drain_writes(k)


def kernel(input):
    T = input.shape[0]
    osz = (T - WINDOW) // STRIDE + 1
    D = input.shape[1]
    run = functools.partial(
        pl.kernel,
        mesh=plsc.VectorSubcoreMesh(core_axis_name="c", subcore_axis_name="s"),
        out_type=jax.ShapeDtypeStruct((osz, WINDOW, D), jnp.float32),
        scratch_types=[
            pltpu.VMEM((NB * STRIDE, D), jnp.float32),
            [pltpu.SemaphoreType.DMA] * NB,
            [pltpu.SemaphoreType.DMA] * NB,
        ],
    )(_sliding_window_sc)
    return run(input)
